# Initial kernel scaffold; baseline (speedup 1.0000x reference)
import jax
import jax.numpy as jnp
from jax.experimental import pallas as pl


def kernel(x_address, x_transaction, edge_index, edge_type, train_mask, W_in_a, b_in_a, W_in_t, b_in_t, W_rel, W_root, b_conv, W_cls, b_cls):
    raise NotImplementedError("write your pallas kernel here")


# speedup vs baseline: 6.0284x; 6.0284x over previous
import functools

import jax
import jax.numpy as jnp
from jax import lax
from jax.experimental import pallas as pl
from jax.experimental.pallas import tpu as pltpu
from jax.experimental.pallas import tpu_sc as plsc

F32 = jnp.float32

N_ADDR = 6000
N_TX = 4000
N = N_ADDR + N_TX
E = 320000
H = 128
R = 4
C = 2

NSUB = 16
NCORE = 2
GRP = 128
NGRP = 160
E_PAD = NSUB * NGRP * GRP
CGRP = E_PAD // (NSUB * NCORE * GRP)

RN = R * N
T = 40960
TRASH = RN
ROWS_PER_SUB = T // NSUB
OUT_PER_SUB = 2504
RN_PAD = NSUB * OUT_PER_SUB
CHUNK = 32
NCHUNK = H // CHUNK
ZR = 1280

BLK = 1000
GRID = N // BLK
ABLK = N_ADDR // BLK


def _proj_body(x_ref, wa_ref, wt_ref, ba_ref, bt_ref, o0, o1, o2, o3):
    i = pl.program_id(0)
    use_a = i < ABLK
    w = jnp.where(use_a, wa_ref[...], wt_ref[...])
    b = jnp.where(use_a, ba_ref[...], bt_ref[...])
    h = jnp.dot(x_ref[...], w, preferred_element_type=F32) + b
    for c, o in enumerate((o0, o1, o2, o3)):
        o[...] = h[:, c * CHUNK:(c + 1) * CHUNK]


def _proj(x, wa, wt, ba, bt):
    return pl.pallas_call(
        _proj_body,
        grid=(GRID,),
        in_specs=[
            pl.BlockSpec((BLK, H), lambda i: (i, 0)),
            pl.BlockSpec((H, H), lambda i: (0, 0)),
            pl.BlockSpec((H, H), lambda i: (0, 0)),
            pl.BlockSpec((1, H), lambda i: (0, 0)),
            pl.BlockSpec((1, H), lambda i: (0, 0)),
        ],
        out_specs=[pl.BlockSpec((BLK, CHUNK), lambda i: (i, 0))] * NCHUNK,
        out_shape=[jax.ShapeDtypeStruct((N, CHUNK), F32)] * NCHUNK,
    )(x, wa, wt, ba, bt)


def _cnt_body(didx, ones_h, zeros_h, out, cnt_sp, dbuf, ones_v, zbuf):
    c = lax.axis_index("c")
    s = lax.axis_index("s")
    pltpu.sync_copy(ones_h, ones_v)
    pltpu.sync_copy(zeros_h, zbuf)
    pltpu.sync_copy(zbuf, cnt_sp.at[pl.ds(s * ROWS_PER_SUB, ROWS_PER_SUB)])
    plsc.subcore_barrier()
    w = c * NSUB + s

    def body(g, carry):
        off = (w * CGRP + g) * GRP
        pltpu.sync_copy(didx.at[pl.ds(off, GRP)], dbuf)
        pltpu.sync_copy(ones_v, cnt_sp.at[dbuf], add=True)
        return carry

    lax.fori_loop(0, CGRP, body, 0)
    plsc.subcore_barrier()
    pltpu.sync_copy(
        cnt_sp.at[pl.ds(s * OUT_PER_SUB, OUT_PER_SUB)],
        out.at[pl.ds(c * RN_PAD + s * OUT_PER_SUB, OUT_PER_SUB)])


def _counts(didx, ones8, zeros8):
    k = pl.kernel(
        _cnt_body,
        out_type=jax.ShapeDtypeStruct((NCORE * RN_PAD, 8), F32),
        mesh=plsc.VectorSubcoreMesh(core_axis_name="c", subcore_axis_name="s"),
        compiler_params=pltpu.CompilerParams(use_tc_tiling_on_sc=False),
        scratch_types=[
            pltpu.VMEM_SHARED((T, 8), F32),
            pltpu.VMEM((GRP,), jnp.int32),
            pltpu.VMEM((GRP, 8), F32),
            pltpu.VMEM((ROWS_PER_SUB, 8), F32),
        ],
    )
    return k(didx, ones8, zeros8)


def _agg_body(h0, h1, h2, h3, sidx, didx, zeros_h, a0, a1, a2, a3,
              agg_sp, sbuf, dbuf, rows, zbuf, sem):
    c = lax.axis_index("c")
    s = lax.axis_index("s")
    pltpu.sync_copy(zeros_h, zbuf)

    def do_pass(table, out):
        for z in range(ROWS_PER_SUB // ZR):
            pltpu.sync_copy(
                zbuf, agg_sp.at[pl.ds(s * ROWS_PER_SUB + z * ZR, ZR)])
        plsc.subcore_barrier()

        def body(g, carry):
            off = (s * NGRP + g) * GRP
            pltpu.sync_copy(sidx.at[pl.ds(off, GRP)], sbuf)
            pltpu.sync_copy(didx.at[pl.ds(off, GRP)], dbuf)
            pltpu.async_copy(table.at[sbuf], rows, sem).wait()
            pltpu.sync_copy(rows, agg_sp.at[dbuf], add=True)
            return carry

        lax.fori_loop(0, NGRP, body, 0)
        plsc.subcore_barrier()
        pltpu.sync_copy(
            agg_sp.at[pl.ds(s * OUT_PER_SUB, OUT_PER_SUB)],
            out.at[pl.ds(s * OUT_PER_SUB, OUT_PER_SUB)])

    @pl.when(c == 0)
    def _():
        do_pass(h0, a0)
        do_pass(h1, a1)

    @pl.when(c == 1)
    def _():
        do_pass(h2, a2)
        do_pass(h3, a3)


def _aggregate(hc, sidx, didx, zeros32):
    k = pl.kernel(
        _agg_body,
        out_type=[jax.ShapeDtypeStruct((RN_PAD, CHUNK), F32)] * NCHUNK,
        mesh=plsc.VectorSubcoreMesh(core_axis_name="c", subcore_axis_name="s"),
        compiler_params=pltpu.CompilerParams(use_tc_tiling_on_sc=False),
        scratch_types=[
            pltpu.VMEM_SHARED((T, CHUNK), F32),
            pltpu.VMEM((GRP,), jnp.int32),
            pltpu.VMEM((GRP,), jnp.int32),
            pltpu.VMEM((GRP, CHUNK), F32),
            pltpu.VMEM((ZR, CHUNK), F32),
            pltpu.SemaphoreType.DMA,
        ],
    )
    return k(hc[0], hc[1], hc[2], hc[3], sidx, didx, zeros32)


def _layer_body(is_last, *refs):
    if is_last:
        (h0, h1, h2, h3, a0, a1, a2, a3, cnt, wroot, wrel, bconv,
         maskf, wcls, bcls, o0, o1, o2, o3, olog) = refs
    else:
        (h0, h1, h2, h3, a0, a1, a2, a3, cnt, wroot, wrel, bconv,
         o0, o1, o2, o3) = refs
    h = jnp.concatenate([h0[...], h1[...], h2[...], h3[...]], axis=1)
    acc = jnp.dot(h, wroot[...], preferred_element_type=F32) + bconv[...]
    wr = wrel[...]
    cn = cnt[...]
    aggs = (a0, a1, a2, a3)
    for r in range(R):
        ar = jnp.concatenate([aggs[c][r] for c in range(NCHUNK)], axis=1)
        m = jnp.dot(ar, wr[r], preferred_element_type=F32)
        ctot = cn[0, r, :, 0:1] + cn[1, r, :, 0:1]
        acc = acc + m * (1.0 / jnp.maximum(ctot, 1.0))
    hn = jnp.maximum(acc, 0.0)
    for c, o in enumerate((o0, o1, o2, o3)):
        o[...] = hn[:, c * CHUNK:(c + 1) * CHUNK]
    if is_last:
        hm = hn * maskf[:, 0:1]
        olog[...] = jnp.dot(hm, wcls[...], preferred_element_type=F32) + bcls[...]


def _layer(hc, aggs, cnt, wroot, wrel, bconv, is_last,
           maskf=None, wcls=None, bcls=None):
    in_specs = (
        [pl.BlockSpec((BLK, CHUNK), lambda i: (i, 0))] * NCHUNK
        + [pl.BlockSpec((R, BLK, CHUNK), lambda i: (0, i, 0))] * NCHUNK
        + [pl.BlockSpec((NCORE, R, BLK, 8), lambda i: (0, 0, i, 0)),
           pl.BlockSpec((H, H), lambda i: (0, 0)),
           pl.BlockSpec((R, H, H), lambda i: (0, 0, 0)),
           pl.BlockSpec((1, H), lambda i: (0, 0))])
    out_specs = [pl.BlockSpec((BLK, CHUNK), lambda i: (i, 0))] * NCHUNK
    out_shape = [jax.ShapeDtypeStruct((N, CHUNK), F32)] * NCHUNK
    args = list(hc) + list(aggs) + [cnt, wroot, wrel, bconv]
    if is_last:
        in_specs += [pl.BlockSpec((BLK, 8), lambda i: (i, 0)),
                     pl.BlockSpec((H, H), lambda i: (0, 0)),
                     pl.BlockSpec((1, H), lambda i: (0, 0))]
        out_specs.append(pl.BlockSpec((BLK, H), lambda i: (i, 0)))
        out_shape.append(jax.ShapeDtypeStruct((N, H), F32))
        args += [maskf, wcls, bcls]
    return pl.pallas_call(
        functools.partial(_layer_body, is_last),
        grid=(GRID,),
        in_specs=in_specs,
        out_specs=out_specs,
        out_shape=out_shape,
    )(*args)


def kernel(x_address, x_transaction, edge_index, edge_type, train_mask,
           W_in_a, b_in_a, W_in_t, b_in_t, W_rel, W_root, b_conv,
           W_cls, b_cls):
    x = jnp.concatenate([x_address, x_transaction], axis=0)
    src = edge_index[0]
    dst = edge_index[1]
    pad = E_PAD - E
    sidx = jnp.concatenate([src, jnp.zeros((pad,), jnp.int32)])
    didx = jnp.concatenate([edge_type * N + dst,
                            jnp.full((pad,), TRASH, jnp.int32)])

    zeros32 = jnp.zeros((ZR, CHUNK), F32)
    zeros8 = jnp.zeros((ROWS_PER_SUB, 8), F32)
    ones8 = jnp.ones((GRP, 8), F32)
    maskf = jnp.concatenate(
        [train_mask[:N_ADDR].astype(F32), jnp.zeros((N - N_ADDR,), F32)]
    )[:, None] * jnp.ones((1, 8), F32)
    wcls_pad = jnp.pad(W_cls, ((0, 0), (0, H - C)))
    bcls_pad = jnp.pad(b_cls, (0, H - C))[None, :]

    hc = _proj(x, W_in_a, W_in_t, b_in_a[None, :], b_in_t[None, :])
    cnt_raw = _counts(didx, ones8, zeros8)
    cnt = jnp.stack([cnt_raw[c * RN_PAD:c * RN_PAD + RN]
                     for c in range(NCORE)]).reshape(NCORE, R, N, 8)

    logits = None
    for l in range(W_root.shape[0]):
        is_last = l == W_root.shape[0] - 1
        aggs = _aggregate(hc, sidx, didx, zeros32)
        aggs = [a[:RN].reshape(R, N, CHUNK) for a in aggs]
        res = _layer(hc, aggs, cnt, W_root[l], W_rel[l], b_conv[l][None, :],
                     is_last, maskf=maskf if is_last else None,
                     wcls=wcls_pad if is_last else None,
                     bcls=bcls_pad if is_last else None)
        if is_last:
            hc, logits = res[:NCHUNK], res[NCHUNK]
        else:
            hc = res
    return logits[:N_ADDR, :C]


# speedup vs baseline: 7.4828x; 1.2413x over previous
import functools

import jax
import jax.numpy as jnp
from jax import lax
from jax.experimental import pallas as pl
from jax.experimental.pallas import tpu as pltpu
from jax.experimental.pallas import tpu_sc as plsc

F32 = jnp.float32

N_ADDR = 6000
N_TX = 4000
N = N_ADDR + N_TX
E = 320000
H = 128
R = 4
C = 2

NSUB = 16
NCORE = 2
GRP = 128
NGRP = 160
KGRP = 8
E_PAD = NSUB * NGRP * GRP
CGRP = E_PAD // (NSUB * NCORE * GRP)

RN = R * N
T = 40960
TRASH = RN
ROWS_PER_SUB = T // NSUB
OUT_PER_SUB = 2504
RN_PAD = NSUB * OUT_PER_SUB
CHUNK = 32
NCHUNK = H // CHUNK
ZR = 160

BLK = 1000
GRID = N // BLK
ABLK = N_ADDR // BLK


def _proj_body(x_ref, wa_ref, wt_ref, ba_ref, bt_ref, o0, o1, o2, o3):
    i = pl.program_id(0)
    use_a = i < ABLK
    w = jnp.where(use_a, wa_ref[...], wt_ref[...])
    b = jnp.where(use_a, ba_ref[...], bt_ref[...])
    h = jnp.dot(x_ref[...], w, preferred_element_type=F32) + b
    for c, o in enumerate((o0, o1, o2, o3)):
        o[...] = h[:, c * CHUNK:(c + 1) * CHUNK]


def _proj(x, wa, wt, ba, bt):
    return pl.pallas_call(
        _proj_body,
        grid=(GRID,),
        in_specs=[
            pl.BlockSpec((BLK, H), lambda i: (i, 0)),
            pl.BlockSpec((H, H), lambda i: (0, 0)),
            pl.BlockSpec((H, H), lambda i: (0, 0)),
            pl.BlockSpec((1, H), lambda i: (0, 0)),
            pl.BlockSpec((1, H), lambda i: (0, 0)),
        ],
        out_specs=[pl.BlockSpec((BLK, CHUNK), lambda i: (i, 0))] * NCHUNK,
        out_shape=[jax.ShapeDtypeStruct((N, CHUNK), F32)] * NCHUNK,
    )(x, wa, wt, ba, bt)


def _cnt_body(didx, ones_h, zeros_h, out, cnt_sp, dbuf, ones_v, zbuf):
    c = lax.axis_index("c")
    s = lax.axis_index("s")
    pltpu.sync_copy(ones_h, ones_v)
    pltpu.sync_copy(zeros_h, zbuf)
    pltpu.sync_copy(zbuf, cnt_sp.at[pl.ds(s * ROWS_PER_SUB, ROWS_PER_SUB)])
    plsc.subcore_barrier()
    w = c * NSUB + s

    def body(g, carry):
        off = (w * CGRP + g) * GRP
        pltpu.sync_copy(didx.at[pl.ds(off, GRP)], dbuf)
        pltpu.sync_copy(ones_v, cnt_sp.at[dbuf], add=True)
        return carry

    lax.fori_loop(0, CGRP, body, 0)
    plsc.subcore_barrier()
    pltpu.sync_copy(
        cnt_sp.at[pl.ds(s * OUT_PER_SUB, OUT_PER_SUB)],
        out.at[pl.ds(c * RN_PAD + s * OUT_PER_SUB, OUT_PER_SUB)])


def _counts(didx, ones8, zeros8):
    k = pl.kernel(
        _cnt_body,
        out_type=jax.ShapeDtypeStruct((NCORE * RN_PAD, 8), F32),
        mesh=plsc.VectorSubcoreMesh(core_axis_name="c", subcore_axis_name="s"),
        compiler_params=pltpu.CompilerParams(use_tc_tiling_on_sc=False),
        scratch_types=[
            pltpu.VMEM_SHARED((T, 8), F32),
            pltpu.VMEM((GRP,), jnp.int32),
            pltpu.VMEM((GRP, 8), F32),
            pltpu.VMEM((ROWS_PER_SUB, 8), F32),
        ],
    )
    return k(didx, ones8, zeros8)


def _agg_body(h0, h1, h2, h3, sidx, didx, zeros_h, a0, a1, a2, a3,
              agg_sp, sbuf, dbuf, rows, zbuf, gsem, ssem):
    c = lax.axis_index("c")
    s = lax.axis_index("s")
    pltpu.sync_copy(zeros_h, zbuf)

    def do_pass(table, out):
        for z in range(ROWS_PER_SUB // ZR):
            pltpu.sync_copy(
                zbuf, agg_sp.at[pl.ds(s * ROWS_PER_SUB + z * ZR, ZR)])
        plsc.subcore_barrier()

        def body(g, carry):
            off = (s * NGRP + g * KGRP) * GRP
            for j in range(KGRP):
                pltpu.sync_copy(sidx.at[pl.ds(off + j * GRP, GRP)],
                                sbuf.at[j])
                pltpu.sync_copy(didx.at[pl.ds(off + j * GRP, GRP)],
                                dbuf.at[j])
            gd = [pltpu.async_copy(table.at[sbuf.at[j]],
                                   rows.at[pl.ds(j * GRP, GRP)], gsem)
                  for j in range(KGRP)]
            sd = []
            for j in range(KGRP):
                gd[j].wait()
                sd.append(pltpu.async_copy(rows.at[pl.ds(j * GRP, GRP)],
                                           agg_sp.at[dbuf.at[j]], ssem,
                                           add=True))
            for d in sd:
                d.wait()
            return carry

        lax.fori_loop(0, NGRP // KGRP, body, 0)
        plsc.subcore_barrier()
        pltpu.sync_copy(
            agg_sp.at[pl.ds(s * OUT_PER_SUB, OUT_PER_SUB)],
            out.at[pl.ds(s * OUT_PER_SUB, OUT_PER_SUB)])

    @pl.when(c == 0)
    def _():
        do_pass(h0, a0)
        do_pass(h1, a1)

    @pl.when(c == 1)
    def _():
        do_pass(h2, a2)
        do_pass(h3, a3)


def _aggregate(hc, sidx, didx, zeros32):
    k = pl.kernel(
        _agg_body,
        out_type=[jax.ShapeDtypeStruct((RN_PAD, CHUNK), F32)] * NCHUNK,
        mesh=plsc.VectorSubcoreMesh(core_axis_name="c", subcore_axis_name="s"),
        compiler_params=pltpu.CompilerParams(use_tc_tiling_on_sc=False),
        scratch_types=[
            pltpu.VMEM_SHARED((T, CHUNK), F32),
            pltpu.VMEM((KGRP, GRP), jnp.int32),
            pltpu.VMEM((KGRP, GRP), jnp.int32),
            pltpu.VMEM((KGRP * GRP, CHUNK), F32),
            pltpu.VMEM((ZR, CHUNK), F32),
            pltpu.SemaphoreType.DMA,
            pltpu.SemaphoreType.DMA,
        ],
    )
    return k(hc[0], hc[1], hc[2], hc[3], sidx, didx, zeros32)


def _layer_body(is_last, *refs):
    if is_last:
        (h0, h1, h2, h3, a0, a1, a2, a3, cnt, wroot, wrel, bconv,
         maskf, wcls, bcls, o0, o1, o2, o3, olog) = refs
    else:
        (h0, h1, h2, h3, a0, a1, a2, a3, cnt, wroot, wrel, bconv,
         o0, o1, o2, o3) = refs
    h = jnp.concatenate([h0[...], h1[...], h2[...], h3[...]], axis=1)
    acc = jnp.dot(h, wroot[...], preferred_element_type=F32) + bconv[...]
    wr = wrel[...]
    cn = cnt[...]
    aggs = (a0, a1, a2, a3)
    for r in range(R):
        ar = jnp.concatenate([aggs[c][r] for c in range(NCHUNK)], axis=1)
        m = jnp.dot(ar, wr[r], preferred_element_type=F32)
        ctot = cn[0, r, :, 0:1] + cn[1, r, :, 0:1]
        acc = acc + m * (1.0 / jnp.maximum(ctot, 1.0))
    hn = jnp.maximum(acc, 0.0)
    for c, o in enumerate((o0, o1, o2, o3)):
        o[...] = hn[:, c * CHUNK:(c + 1) * CHUNK]
    if is_last:
        hm = hn * maskf[:, 0:1]
        olog[...] = jnp.dot(hm, wcls[...], preferred_element_type=F32) + bcls[...]


def _layer(hc, aggs, cnt, wroot, wrel, bconv, is_last,
           maskf=None, wcls=None, bcls=None):
    in_specs = (
        [pl.BlockSpec((BLK, CHUNK), lambda i: (i, 0))] * NCHUNK
        + [pl.BlockSpec((R, BLK, CHUNK), lambda i: (0, i, 0))] * NCHUNK
        + [pl.BlockSpec((NCORE, R, BLK, 8), lambda i: (0, 0, i, 0)),
           pl.BlockSpec((H, H), lambda i: (0, 0)),
           pl.BlockSpec((R, H, H), lambda i: (0, 0, 0)),
           pl.BlockSpec((1, H), lambda i: (0, 0))])
    out_specs = [pl.BlockSpec((BLK, CHUNK), lambda i: (i, 0))] * NCHUNK
    out_shape = [jax.ShapeDtypeStruct((N, CHUNK), F32)] * NCHUNK
    args = list(hc) + list(aggs) + [cnt, wroot, wrel, bconv]
    if is_last:
        in_specs += [pl.BlockSpec((BLK, 8), lambda i: (i, 0)),
                     pl.BlockSpec((H, H), lambda i: (0, 0)),
                     pl.BlockSpec((1, H), lambda i: (0, 0))]
        out_specs.append(pl.BlockSpec((BLK, H), lambda i: (i, 0)))
        out_shape.append(jax.ShapeDtypeStruct((N, H), F32))
        args += [maskf, wcls, bcls]
    return pl.pallas_call(
        functools.partial(_layer_body, is_last),
        grid=(GRID,),
        in_specs=in_specs,
        out_specs=out_specs,
        out_shape=out_shape,
    )(*args)


def kernel(x_address, x_transaction, edge_index, edge_type, train_mask,
           W_in_a, b_in_a, W_in_t, b_in_t, W_rel, W_root, b_conv,
           W_cls, b_cls):
    x = jnp.concatenate([x_address, x_transaction], axis=0)
    src = edge_index[0]
    dst = edge_index[1]
    pad = E_PAD - E
    sidx = jnp.concatenate([src, jnp.zeros((pad,), jnp.int32)])
    didx = jnp.concatenate([edge_type * N + dst,
                            jnp.full((pad,), TRASH, jnp.int32)])

    zeros32 = jnp.zeros((ZR, CHUNK), F32)
    zeros8 = jnp.zeros((ROWS_PER_SUB, 8), F32)
    ones8 = jnp.ones((GRP, 8), F32)
    maskf = jnp.concatenate(
        [train_mask[:N_ADDR].astype(F32), jnp.zeros((N - N_ADDR,), F32)]
    )[:, None] * jnp.ones((1, 8), F32)
    wcls_pad = jnp.pad(W_cls, ((0, 0), (0, H - C)))
    bcls_pad = jnp.pad(b_cls, (0, H - C))[None, :]

    hc = _proj(x, W_in_a, W_in_t, b_in_a[None, :], b_in_t[None, :])
    cnt_raw = _counts(didx, ones8, zeros8)
    cnt = jnp.stack([cnt_raw[c * RN_PAD:c * RN_PAD + RN]
                     for c in range(NCORE)]).reshape(NCORE, R, N, 8)

    logits = None
    for l in range(W_root.shape[0]):
        is_last = l == W_root.shape[0] - 1
        aggs = _aggregate(hc, sidx, didx, zeros32)
        aggs = [a[:RN].reshape(R, N, CHUNK) for a in aggs]
        res = _layer(hc, aggs, cnt, W_root[l], W_rel[l], b_conv[l][None, :],
                     is_last, maskf=maskf if is_last else None,
                     wcls=wcls_pad if is_last else None,
                     bcls=bcls_pad if is_last else None)
        if is_last:
            hc, logits = res[:NCHUNK], res[NCHUNK]
        else:
            hc = res
    return logits[:N_ADDR, :C]


# speedup vs baseline: 10.1504x; 1.3565x over previous
import functools

import jax
import jax.numpy as jnp
from jax import lax
from jax.experimental import pallas as pl
from jax.experimental.pallas import tpu as pltpu
from jax.experimental.pallas import tpu_sc as plsc

F32 = jnp.float32

N_ADDR = 6000
N_TX = 4000
N = N_ADDR + N_TX
E = 320000
H = 128
R = 4
C = 2

NSUB = 16
NCORE = 2
GRP = 128
NGRP = 160
KGRP = 8
E_PAD = NSUB * NGRP * GRP
CGRP = E_PAD // (NSUB * NCORE * GRP)

RN = R * N
T = 40960
TRASH = RN
ROWS_PER_SUB = T // NSUB
OUT_PER_SUB = 2504
RN_PAD = NSUB * OUT_PER_SUB
CHUNK = 32
NCHUNK = H // CHUNK
ZR = 160

BLK = 1000
GRID = N // BLK
ABLK = N_ADDR // BLK


def _proj_body(x_ref, wa_ref, wt_ref, ba_ref, bt_ref, o0, o1, o2, o3):
    i = pl.program_id(0)
    use_a = i < ABLK
    w = jnp.where(use_a, wa_ref[...], wt_ref[...])
    b = jnp.where(use_a, ba_ref[...], bt_ref[...])
    h = jnp.dot(x_ref[...], w, preferred_element_type=F32) + b
    for c, o in enumerate((o0, o1, o2, o3)):
        o[...] = h[:, c * CHUNK:(c + 1) * CHUNK]


def _proj(x, wa, wt, ba, bt):
    return pl.pallas_call(
        _proj_body,
        grid=(GRID,),
        in_specs=[
            pl.BlockSpec((BLK, H), lambda i: (i, 0)),
            pl.BlockSpec((H, H), lambda i: (0, 0)),
            pl.BlockSpec((H, H), lambda i: (0, 0)),
            pl.BlockSpec((1, H), lambda i: (0, 0)),
            pl.BlockSpec((1, H), lambda i: (0, 0)),
        ],
        out_specs=[pl.BlockSpec((BLK, CHUNK), lambda i: (i, 0))] * NCHUNK,
        out_shape=[jax.ShapeDtypeStruct((N, CHUNK), F32)] * NCHUNK,
    )(x, wa, wt, ba, bt)


def _cnt_body(didx, ones_h, zeros_h, out, cnt_sp, dbuf, ones_v, zbuf):
    c = lax.axis_index("c")
    s = lax.axis_index("s")
    pltpu.sync_copy(ones_h, ones_v)
    pltpu.sync_copy(zeros_h, zbuf)
    pltpu.sync_copy(zbuf, cnt_sp.at[pl.ds(s * ROWS_PER_SUB, ROWS_PER_SUB)])
    plsc.subcore_barrier()
    w = c * NSUB + s

    def body(g, carry):
        off = (w * CGRP + g) * GRP
        pltpu.sync_copy(didx.at[pl.ds(off, GRP)], dbuf)
        pltpu.sync_copy(ones_v, cnt_sp.at[dbuf], add=True)
        return carry

    lax.fori_loop(0, CGRP, body, 0)
    plsc.subcore_barrier()
    pltpu.sync_copy(
        cnt_sp.at[pl.ds(s * OUT_PER_SUB, OUT_PER_SUB)],
        out.at[pl.ds(c * RN_PAD + s * OUT_PER_SUB, OUT_PER_SUB)])


def _counts(didx, ones8, zeros8):
    k = pl.kernel(
        _cnt_body,
        out_type=jax.ShapeDtypeStruct((NCORE * RN_PAD, 8), F32),
        mesh=plsc.VectorSubcoreMesh(core_axis_name="c", subcore_axis_name="s"),
        compiler_params=pltpu.CompilerParams(use_tc_tiling_on_sc=False),
        scratch_types=[
            pltpu.VMEM_SHARED((T, 8), F32),
            pltpu.VMEM((GRP,), jnp.int32),
            pltpu.VMEM((GRP, 8), F32),
            pltpu.VMEM((ROWS_PER_SUB, 8), F32),
        ],
    )
    return k(didx, ones8, zeros8)


def _agg_body(h0, h1, h2, h3, sidx, didx, zeros_h, a0, a1, a2, a3,
              agg_sp, sbuf, dbuf, rows, zbuf, gsem, ssem):
    c = lax.axis_index("c")
    s = lax.axis_index("s")
    pltpu.sync_copy(zeros_h, zbuf)

    def do_pass(table, out):
        for z in range(ROWS_PER_SUB // ZR):
            pltpu.sync_copy(
                zbuf, agg_sp.at[pl.ds(s * ROWS_PER_SUB + z * ZR, ZR)])
        plsc.subcore_barrier()

        def body(g, carry):
            row0 = s * NGRP + g * KGRP
            pltpu.sync_copy(sidx.at[pl.ds(row0, KGRP)], sbuf)
            pltpu.sync_copy(didx.at[pl.ds(row0, KGRP)], dbuf)
            gd = [pltpu.async_copy(table.at[sbuf.at[j]],
                                   rows.at[pl.ds(j * GRP, GRP)], gsem)
                  for j in range(KGRP)]
            sd = []
            for j in range(KGRP):
                gd[j].wait()
                sd.append(pltpu.async_copy(rows.at[pl.ds(j * GRP, GRP)],
                                           agg_sp.at[dbuf.at[j]], ssem,
                                           add=True))
            for d in sd:
                d.wait()
            return carry

        lax.fori_loop(0, NGRP // KGRP, body, 0)
        plsc.subcore_barrier()
        pltpu.sync_copy(
            agg_sp.at[pl.ds(s * OUT_PER_SUB, OUT_PER_SUB)],
            out.at[pl.ds(s * OUT_PER_SUB, OUT_PER_SUB)])

    @pl.when(c == 0)
    def _():
        do_pass(h0, a0)
        do_pass(h1, a1)

    @pl.when(c == 1)
    def _():
        do_pass(h2, a2)
        do_pass(h3, a3)


def _aggregate(hc, sidx, didx, zeros32):
    k = pl.kernel(
        _agg_body,
        out_type=[jax.ShapeDtypeStruct((RN_PAD, CHUNK), F32)] * NCHUNK,
        mesh=plsc.VectorSubcoreMesh(core_axis_name="c", subcore_axis_name="s"),
        compiler_params=pltpu.CompilerParams(use_tc_tiling_on_sc=False),
        scratch_types=[
            pltpu.VMEM_SHARED((T, CHUNK), F32),
            pltpu.VMEM((KGRP, GRP), jnp.int32),
            pltpu.VMEM((KGRP, GRP), jnp.int32),
            pltpu.VMEM((KGRP * GRP, CHUNK), F32),
            pltpu.VMEM((ZR, CHUNK), F32),
            pltpu.SemaphoreType.DMA,
            pltpu.SemaphoreType.DMA,
        ],
    )
    return k(hc[0], hc[1], hc[2], hc[3], sidx, didx, zeros32)


def _layer_body(is_last, *refs):
    if is_last:
        (h0, h1, h2, h3, a0, a1, a2, a3, cnt, wroot, wrel, bconv,
         maskf, wcls, bcls, o0, o1, o2, o3, olog) = refs
    else:
        (h0, h1, h2, h3, a0, a1, a2, a3, cnt, wroot, wrel, bconv,
         o0, o1, o2, o3) = refs
    h = jnp.concatenate([h0[...], h1[...], h2[...], h3[...]], axis=1)
    acc = jnp.dot(h, wroot[...], preferred_element_type=F32) + bconv[...]
    wr = wrel[...]
    cn = cnt[...]
    aggs = (a0, a1, a2, a3)
    for r in range(R):
        ar = jnp.concatenate([aggs[c][r] for c in range(NCHUNK)], axis=1)
        m = jnp.dot(ar, wr[r], preferred_element_type=F32)
        ctot = cn[0, r, :, 0:1] + cn[1, r, :, 0:1]
        acc = acc + m * (1.0 / jnp.maximum(ctot, 1.0))
    hn = jnp.maximum(acc, 0.0)
    for c, o in enumerate((o0, o1, o2, o3)):
        o[...] = hn[:, c * CHUNK:(c + 1) * CHUNK]
    if is_last:
        hm = hn * maskf[:, 0:1]
        olog[...] = jnp.dot(hm, wcls[...], preferred_element_type=F32) + bcls[...]


def _layer(hc, aggs, cnt, wroot, wrel, bconv, is_last,
           maskf=None, wcls=None, bcls=None):
    in_specs = (
        [pl.BlockSpec((BLK, CHUNK), lambda i: (i, 0))] * NCHUNK
        + [pl.BlockSpec((R, BLK, CHUNK), lambda i: (0, i, 0))] * NCHUNK
        + [pl.BlockSpec((NCORE, R, BLK, 8), lambda i: (0, 0, i, 0)),
           pl.BlockSpec((H, H), lambda i: (0, 0)),
           pl.BlockSpec((R, H, H), lambda i: (0, 0, 0)),
           pl.BlockSpec((1, H), lambda i: (0, 0))])
    out_specs = [pl.BlockSpec((BLK, CHUNK), lambda i: (i, 0))] * NCHUNK
    out_shape = [jax.ShapeDtypeStruct((N, CHUNK), F32)] * NCHUNK
    args = list(hc) + list(aggs) + [cnt, wroot, wrel, bconv]
    if is_last:
        in_specs += [pl.BlockSpec((BLK, 8), lambda i: (i, 0)),
                     pl.BlockSpec((H, H), lambda i: (0, 0)),
                     pl.BlockSpec((1, H), lambda i: (0, 0))]
        out_specs.append(pl.BlockSpec((BLK, H), lambda i: (i, 0)))
        out_shape.append(jax.ShapeDtypeStruct((N, H), F32))
        args += [maskf, wcls, bcls]
    return pl.pallas_call(
        functools.partial(_layer_body, is_last),
        grid=(GRID,),
        in_specs=in_specs,
        out_specs=out_specs,
        out_shape=out_shape,
    )(*args)


def kernel(x_address, x_transaction, edge_index, edge_type, train_mask,
           W_in_a, b_in_a, W_in_t, b_in_t, W_rel, W_root, b_conv,
           W_cls, b_cls):
    x = jnp.concatenate([x_address, x_transaction], axis=0)
    src = edge_index[0]
    dst = edge_index[1]
    pad = E_PAD - E
    sidx = jnp.concatenate([src, jnp.zeros((pad,), jnp.int32)])
    didx = jnp.concatenate([edge_type * N + dst,
                            jnp.full((pad,), TRASH, jnp.int32)])

    zeros32 = jnp.zeros((ZR, CHUNK), F32)
    zeros8 = jnp.zeros((ROWS_PER_SUB, 8), F32)
    ones8 = jnp.ones((GRP, 8), F32)
    maskf = jnp.concatenate(
        [train_mask[:N_ADDR].astype(F32), jnp.zeros((N - N_ADDR,), F32)]
    )[:, None] * jnp.ones((1, 8), F32)
    wcls_pad = jnp.pad(W_cls, ((0, 0), (0, H - C)))
    bcls_pad = jnp.pad(b_cls, (0, H - C))[None, :]

    sidx2 = sidx.reshape(E_PAD // GRP, GRP)
    didx2 = didx.reshape(E_PAD // GRP, GRP)

    hc = _proj(x, W_in_a, W_in_t, b_in_a[None, :], b_in_t[None, :])
    cnt_raw = _counts(didx, ones8, zeros8)
    cnt = jnp.stack([cnt_raw[c * RN_PAD:c * RN_PAD + RN]
                     for c in range(NCORE)]).reshape(NCORE, R, N, 8)

    logits = None
    for l in range(W_root.shape[0]):
        is_last = l == W_root.shape[0] - 1
        aggs = _aggregate(hc, sidx2, didx2, zeros32)
        aggs = [a[:RN].reshape(R, N, CHUNK) for a in aggs]
        res = _layer(hc, aggs, cnt, W_root[l], W_rel[l], b_conv[l][None, :],
                     is_last, maskf=maskf if is_last else None,
                     wcls=wcls_pad if is_last else None,
                     bcls=bcls_pad if is_last else None)
        if is_last:
            hc, logits = res[:NCHUNK], res[NCHUNK]
        else:
            hc = res
    return logits[:N_ADDR, :C]


# speedup vs baseline: 14.6888x; 1.4471x over previous
import functools

import jax
import jax.numpy as jnp
from jax import lax
from jax.experimental import pallas as pl
from jax.experimental.pallas import tpu as pltpu
from jax.experimental.pallas import tpu_sc as plsc

F32 = jnp.float32
BF16 = jnp.bfloat16

N_ADDR = 6000
N_TX = 4000
N = N_ADDR + N_TX
E = 320000
H = 128
R = 4
C = 2

NSUB = 16
NCORE = 2
GRP = 128
NGRP = 160
KGRP = 8
E_PAD = NSUB * NGRP * GRP
CGRP = E_PAD // (NSUB * NCORE * GRP)

RN = R * N
T = 40960
TRASH = RN
ROWS_PER_SUB = T // NSUB
OUT_PER_SUB = 2504
RN_PAD = NSUB * OUT_PER_SUB
CHUNK = 32
NCHUNK = H // CHUNK
ZR = 160

BLK = 1000
GRID = N // BLK
ABLK = N_ADDR // BLK


def _proj_body(x_ref, wa_ref, wt_ref, ba_ref, bt_ref, o0, o1, o2, o3):
    i = pl.program_id(0)
    use_a = i < ABLK
    w = jnp.where(use_a, wa_ref[...], wt_ref[...])
    b = jnp.where(use_a, ba_ref[...], bt_ref[...])
    h = (jnp.dot(x_ref[...], w, preferred_element_type=F32) + b).astype(BF16)
    for c, o in enumerate((o0, o1, o2, o3)):
        o[...] = h[:, c * CHUNK:(c + 1) * CHUNK]


def _proj(x, wa, wt, ba, bt):
    return pl.pallas_call(
        _proj_body,
        grid=(GRID,),
        in_specs=[
            pl.BlockSpec((BLK, H), lambda i: (i, 0)),
            pl.BlockSpec((H, H), lambda i: (0, 0)),
            pl.BlockSpec((H, H), lambda i: (0, 0)),
            pl.BlockSpec((1, H), lambda i: (0, 0)),
            pl.BlockSpec((1, H), lambda i: (0, 0)),
        ],
        out_specs=[pl.BlockSpec((BLK, CHUNK), lambda i: (i, 0))] * NCHUNK,
        out_shape=[jax.ShapeDtypeStruct((N, CHUNK), BF16)] * NCHUNK,
    )(x, wa, wt, ba, bt)


def _cnt_body(didx, ones_h, zeros_h, out, cnt_sp, dbuf, ones_v, zbuf):
    c = lax.axis_index("c")
    s = lax.axis_index("s")
    pltpu.sync_copy(ones_h, ones_v)
    pltpu.sync_copy(zeros_h, zbuf)
    pltpu.sync_copy(zbuf, cnt_sp.at[pl.ds(s * ROWS_PER_SUB, ROWS_PER_SUB)])
    plsc.subcore_barrier()
    w = c * NSUB + s

    def body(g, carry):
        off = (w * CGRP + g) * GRP
        pltpu.sync_copy(didx.at[pl.ds(off, GRP)], dbuf)
        pltpu.sync_copy(ones_v, cnt_sp.at[dbuf], add=True)
        return carry

    lax.fori_loop(0, CGRP, body, 0)
    plsc.subcore_barrier()
    pltpu.sync_copy(
        cnt_sp.at[pl.ds(s * OUT_PER_SUB, OUT_PER_SUB)],
        out.at[pl.ds(c * RN_PAD + s * OUT_PER_SUB, OUT_PER_SUB)])


def _counts(didx, ones8, zeros8):
    k = pl.kernel(
        _cnt_body,
        out_type=jax.ShapeDtypeStruct((NCORE * RN_PAD, 8), F32),
        mesh=plsc.VectorSubcoreMesh(core_axis_name="c", subcore_axis_name="s"),
        compiler_params=pltpu.CompilerParams(use_tc_tiling_on_sc=False),
        scratch_types=[
            pltpu.VMEM_SHARED((T, 8), F32),
            pltpu.VMEM((GRP,), jnp.int32),
            pltpu.VMEM((GRP, 8), F32),
            pltpu.VMEM((ROWS_PER_SUB, 8), F32),
        ],
    )
    return k(didx, ones8, zeros8)


def _agg_body(h0, h1, h2, h3, sidx, didx, zeros_h, a0, a1, a2, a3,
              agg_sp, sbuf, dbuf, rows, zbuf, gsem, ssem):
    c = lax.axis_index("c")
    s = lax.axis_index("s")
    pltpu.sync_copy(zeros_h, zbuf)

    def do_pass(table, out):
        for z in range(ROWS_PER_SUB // ZR):
            pltpu.sync_copy(
                zbuf, agg_sp.at[pl.ds(s * ROWS_PER_SUB + z * ZR, ZR)])
        plsc.subcore_barrier()

        def body(g, carry):
            row0 = s * NGRP + g * KGRP
            pltpu.sync_copy(sidx.at[pl.ds(row0, KGRP)], sbuf)
            pltpu.sync_copy(didx.at[pl.ds(row0, KGRP)], dbuf)
            gd = [pltpu.async_copy(table.at[sbuf.at[j]],
                                   rows.at[pl.ds(j * GRP, GRP)], gsem)
                  for j in range(KGRP)]
            sd = []
            for j in range(KGRP):
                gd[j].wait()
                sd.append(pltpu.async_copy(rows.at[pl.ds(j * GRP, GRP)],
                                           agg_sp.at[dbuf.at[j]], ssem,
                                           add=True))
            for d in sd:
                d.wait()
            return carry

        lax.fori_loop(0, NGRP // KGRP, body, 0)
        plsc.subcore_barrier()
        pltpu.sync_copy(
            agg_sp.at[pl.ds(s * OUT_PER_SUB, OUT_PER_SUB)],
            out.at[pl.ds(s * OUT_PER_SUB, OUT_PER_SUB)])

    @pl.when(c == 0)
    def _():
        do_pass(h0, a0)
        do_pass(h1, a1)

    @pl.when(c == 1)
    def _():
        do_pass(h2, a2)
        do_pass(h3, a3)


def _aggregate(hc, sidx, didx, zeros32):
    k = pl.kernel(
        _agg_body,
        out_type=[jax.ShapeDtypeStruct((RN_PAD, CHUNK), BF16)] * NCHUNK,
        mesh=plsc.VectorSubcoreMesh(core_axis_name="c", subcore_axis_name="s"),
        compiler_params=pltpu.CompilerParams(use_tc_tiling_on_sc=False),
        scratch_types=[
            pltpu.VMEM_SHARED((T, CHUNK), BF16),
            pltpu.VMEM((KGRP, GRP), jnp.int32),
            pltpu.VMEM((KGRP, GRP), jnp.int32),
            pltpu.VMEM((KGRP * GRP, CHUNK), BF16),
            pltpu.VMEM((ZR, CHUNK), BF16),
            pltpu.SemaphoreType.DMA,
            pltpu.SemaphoreType.DMA,
        ],
    )
    return k(hc[0], hc[1], hc[2], hc[3], sidx, didx, zeros32)


def _layer_body(is_last, *refs):
    if is_last:
        (h0, h1, h2, h3, a0, a1, a2, a3, cnt, wroot, wrel, bconv,
         maskf, wcls, bcls, o0, o1, o2, o3, olog) = refs
    else:
        (h0, h1, h2, h3, a0, a1, a2, a3, cnt, wroot, wrel, bconv,
         o0, o1, o2, o3) = refs
    h = jnp.concatenate([h0[...], h1[...], h2[...], h3[...]],
                        axis=1).astype(F32)
    acc = jnp.dot(h, wroot[...], preferred_element_type=F32) + bconv[...]
    wr = wrel[...]
    cn = cnt[...]
    aggs = (a0, a1, a2, a3)
    for r in range(R):
        ar = jnp.concatenate([aggs[c][r] for c in range(NCHUNK)],
                             axis=1).astype(F32)
        m = jnp.dot(ar, wr[r], preferred_element_type=F32)
        ctot = cn[0, r, :, 0:1] + cn[1, r, :, 0:1]
        acc = acc + m * (1.0 / jnp.maximum(ctot, 1.0))
    hn = jnp.maximum(acc, 0.0)
    hb = hn.astype(BF16)
    for c, o in enumerate((o0, o1, o2, o3)):
        o[...] = hb[:, c * CHUNK:(c + 1) * CHUNK]
    if is_last:
        hm = hn * maskf[:, 0:1]
        olog[...] = jnp.dot(hm, wcls[...], preferred_element_type=F32) + bcls[...]


def _layer(hc, aggs, cnt, wroot, wrel, bconv, is_last,
           maskf=None, wcls=None, bcls=None):
    in_specs = (
        [pl.BlockSpec((BLK, CHUNK), lambda i: (i, 0))] * NCHUNK
        + [pl.BlockSpec((R, BLK, CHUNK), lambda i: (0, i, 0))] * NCHUNK
        + [pl.BlockSpec((NCORE, R, BLK, 8), lambda i: (0, 0, i, 0)),
           pl.BlockSpec((H, H), lambda i: (0, 0)),
           pl.BlockSpec((R, H, H), lambda i: (0, 0, 0)),
           pl.BlockSpec((1, H), lambda i: (0, 0))])
    out_specs = [pl.BlockSpec((BLK, CHUNK), lambda i: (i, 0))] * NCHUNK
    out_shape = [jax.ShapeDtypeStruct((N, CHUNK), BF16)] * NCHUNK
    args = list(hc) + list(aggs) + [cnt, wroot, wrel, bconv]
    if is_last:
        in_specs += [pl.BlockSpec((BLK, 8), lambda i: (i, 0)),
                     pl.BlockSpec((H, H), lambda i: (0, 0)),
                     pl.BlockSpec((1, H), lambda i: (0, 0))]
        out_specs.append(pl.BlockSpec((BLK, H), lambda i: (i, 0)))
        out_shape.append(jax.ShapeDtypeStruct((N, H), F32))
        args += [maskf, wcls, bcls]
    return pl.pallas_call(
        functools.partial(_layer_body, is_last),
        grid=(GRID,),
        in_specs=in_specs,
        out_specs=out_specs,
        out_shape=out_shape,
    )(*args)


def kernel(x_address, x_transaction, edge_index, edge_type, train_mask,
           W_in_a, b_in_a, W_in_t, b_in_t, W_rel, W_root, b_conv,
           W_cls, b_cls):
    x = jnp.concatenate([x_address, x_transaction], axis=0)
    src = edge_index[0]
    dst = edge_index[1]
    pad = E_PAD - E
    sidx = jnp.concatenate([src, jnp.zeros((pad,), jnp.int32)])
    didx = jnp.concatenate([edge_type * N + dst,
                            jnp.full((pad,), TRASH, jnp.int32)])

    zeros32 = jnp.zeros((ZR, CHUNK), BF16)
    zeros8 = jnp.zeros((ROWS_PER_SUB, 8), F32)
    ones8 = jnp.ones((GRP, 8), F32)
    maskf = jnp.concatenate(
        [train_mask[:N_ADDR].astype(F32), jnp.zeros((N - N_ADDR,), F32)]
    )[:, None] * jnp.ones((1, 8), F32)
    wcls_pad = jnp.pad(W_cls, ((0, 0), (0, H - C)))
    bcls_pad = jnp.pad(b_cls, (0, H - C))[None, :]

    sidx2 = sidx.reshape(E_PAD // GRP, GRP)
    didx2 = didx.reshape(E_PAD // GRP, GRP)

    hc = _proj(x, W_in_a, W_in_t, b_in_a[None, :], b_in_t[None, :])
    cnt_raw = _counts(didx, ones8, zeros8)
    cnt = jnp.stack([cnt_raw[c * RN_PAD:c * RN_PAD + RN]
                     for c in range(NCORE)]).reshape(NCORE, R, N, 8)

    logits = None
    for l in range(W_root.shape[0]):
        is_last = l == W_root.shape[0] - 1
        aggs = _aggregate(hc, sidx2, didx2, zeros32)
        aggs = [a[:RN].reshape(R, N, CHUNK) for a in aggs]
        res = _layer(hc, aggs, cnt, W_root[l], W_rel[l], b_conv[l][None, :],
                     is_last, maskf=maskf if is_last else None,
                     wcls=wcls_pad if is_last else None,
                     bcls=bcls_pad if is_last else None)
        if is_last:
            hc, logits = res[:NCHUNK], res[NCHUNK]
        else:
            hc = res
    return logits[:N_ADDR, :C]


# speedup vs baseline: 17.8168x; 1.2129x over previous
import functools

import jax
import jax.numpy as jnp
from jax import lax
from jax.experimental import pallas as pl
from jax.experimental.pallas import tpu as pltpu
from jax.experimental.pallas import tpu_sc as plsc

F32 = jnp.float32
BF16 = jnp.bfloat16

N_ADDR = 6000
N_TX = 4000
N = N_ADDR + N_TX
E = 320000
H = 128
R = 4
C = 2

NSUB = 16
NCORE = 2
GRP = 128
NGRP = 160
KGRP = 8
E_PAD = NSUB * NGRP * GRP
CGRP = E_PAD // (NSUB * NCORE * GRP)

RN = R * N
T = 40960
TRASH = RN
ROWS_PER_SUB = T // NSUB
OUT_PER_SUB = 2504
RN_PAD = NSUB * OUT_PER_SUB
CHUNK = 64
NCHUNK = H // CHUNK
ZR = 160

BLK = 1000
GRID = N // BLK
ABLK = N_ADDR // BLK


def _proj_body(x_ref, wa_ref, wt_ref, ba_ref, bt_ref, o0, o1):
    i = pl.program_id(0)
    use_a = i < ABLK
    w = jnp.where(use_a, wa_ref[...], wt_ref[...])
    b = jnp.where(use_a, ba_ref[...], bt_ref[...])
    h = (jnp.dot(x_ref[...], w, preferred_element_type=F32) + b).astype(BF16)
    for c, o in enumerate((o0, o1)):
        o[...] = h[:, c * CHUNK:(c + 1) * CHUNK]


def _proj(x, wa, wt, ba, bt):
    return pl.pallas_call(
        _proj_body,
        grid=(GRID,),
        in_specs=[
            pl.BlockSpec((BLK, H), lambda i: (i, 0)),
            pl.BlockSpec((H, H), lambda i: (0, 0)),
            pl.BlockSpec((H, H), lambda i: (0, 0)),
            pl.BlockSpec((1, H), lambda i: (0, 0)),
            pl.BlockSpec((1, H), lambda i: (0, 0)),
        ],
        out_specs=[pl.BlockSpec((BLK, CHUNK), lambda i: (i, 0))] * NCHUNK,
        out_shape=[jax.ShapeDtypeStruct((N, CHUNK), BF16)] * NCHUNK,
    )(x, wa, wt, ba, bt)


def _cnt_body(didx, ones_h, zeros_h, out, cnt_sp, dbuf, ones_v, zbuf):
    c = lax.axis_index("c")
    s = lax.axis_index("s")
    pltpu.sync_copy(ones_h, ones_v)
    pltpu.sync_copy(zeros_h, zbuf)
    pltpu.sync_copy(zbuf, cnt_sp.at[pl.ds(s * ROWS_PER_SUB, ROWS_PER_SUB)])
    plsc.subcore_barrier()
    w = c * NSUB + s

    def body(g, carry):
        off = (w * CGRP + g) * GRP
        pltpu.sync_copy(didx.at[pl.ds(off, GRP)], dbuf)
        pltpu.sync_copy(ones_v, cnt_sp.at[dbuf], add=True)
        return carry

    lax.fori_loop(0, CGRP, body, 0)
    plsc.subcore_barrier()
    pltpu.sync_copy(
        cnt_sp.at[pl.ds(s * OUT_PER_SUB, OUT_PER_SUB)],
        out.at[pl.ds(c * RN_PAD + s * OUT_PER_SUB, OUT_PER_SUB)])


def _counts(didx, ones8, zeros8):
    k = pl.kernel(
        _cnt_body,
        out_type=jax.ShapeDtypeStruct((NCORE * RN_PAD, 8), F32),
        mesh=plsc.VectorSubcoreMesh(core_axis_name="c", subcore_axis_name="s"),
        compiler_params=pltpu.CompilerParams(use_tc_tiling_on_sc=False),
        scratch_types=[
            pltpu.VMEM_SHARED((T, 8), F32),
            pltpu.VMEM((GRP,), jnp.int32),
            pltpu.VMEM((GRP, 8), F32),
            pltpu.VMEM((ROWS_PER_SUB, 8), F32),
        ],
    )
    return k(didx, ones8, zeros8)


def _agg_body(h0, h1, sidx, didx, zeros_h, a0, a1,
              agg_sp, sbuf, dbuf, rows, zbuf, gsem, ssem):
    c = lax.axis_index("c")
    s = lax.axis_index("s")
    pltpu.sync_copy(zeros_h, zbuf)

    def do_pass(table, out):
        for z in range(ROWS_PER_SUB // ZR):
            pltpu.sync_copy(
                zbuf, agg_sp.at[pl.ds(s * ROWS_PER_SUB + z * ZR, ZR)])
        plsc.subcore_barrier()

        def body(g, carry):
            row0 = s * NGRP + g * KGRP
            pltpu.sync_copy(sidx.at[pl.ds(row0, KGRP)], sbuf)
            pltpu.sync_copy(didx.at[pl.ds(row0, KGRP)], dbuf)
            gd = [pltpu.async_copy(table.at[sbuf.at[j]],
                                   rows.at[pl.ds(j * GRP, GRP)], gsem)
                  for j in range(KGRP)]
            sd = []
            for j in range(KGRP):
                gd[j].wait()
                sd.append(pltpu.async_copy(rows.at[pl.ds(j * GRP, GRP)],
                                           agg_sp.at[dbuf.at[j]], ssem,
                                           add=True))
            for d in sd:
                d.wait()
            return carry

        lax.fori_loop(0, NGRP // KGRP, body, 0)
        plsc.subcore_barrier()
        pltpu.sync_copy(
            agg_sp.at[pl.ds(s * OUT_PER_SUB, OUT_PER_SUB)],
            out.at[pl.ds(s * OUT_PER_SUB, OUT_PER_SUB)])

    @pl.when(c == 0)
    def _():
        do_pass(h0, a0)

    @pl.when(c == 1)
    def _():
        do_pass(h1, a1)


def _aggregate(hc, sidx, didx, zeros64):
    k = pl.kernel(
        _agg_body,
        out_type=[jax.ShapeDtypeStruct((RN_PAD, CHUNK), BF16)] * NCHUNK,
        mesh=plsc.VectorSubcoreMesh(core_axis_name="c", subcore_axis_name="s"),
        compiler_params=pltpu.CompilerParams(use_tc_tiling_on_sc=False),
        scratch_types=[
            pltpu.VMEM_SHARED((T, CHUNK), BF16),
            pltpu.VMEM((KGRP, GRP), jnp.int32),
            pltpu.VMEM((KGRP, GRP), jnp.int32),
            pltpu.VMEM((KGRP * GRP, CHUNK), BF16),
            pltpu.VMEM((ZR, CHUNK), BF16),
            pltpu.SemaphoreType.DMA,
            pltpu.SemaphoreType.DMA,
        ],
    )
    return k(hc[0], hc[1], sidx, didx, zeros64)


def _layer_body(is_last, *refs):
    if is_last:
        (h0, h1, a0, a1, cnt, wroot, wrel, bconv,
         maskf, wcls, bcls, o0, o1, olog) = refs
    else:
        (h0, h1, a0, a1, cnt, wroot, wrel, bconv, o0, o1) = refs
    h = jnp.concatenate([h0[...], h1[...]], axis=1).astype(F32)
    acc = jnp.dot(h, wroot[...], preferred_element_type=F32) + bconv[...]
    wr = wrel[...]
    cn = cnt[...]
    aggs = (a0, a1)
    for r in range(R):
        ar = jnp.concatenate([aggs[c][r] for c in range(NCHUNK)],
                             axis=1).astype(F32)
        m = jnp.dot(ar, wr[r], preferred_element_type=F32)
        ctot = cn[0, r, :, 0:1] + cn[1, r, :, 0:1]
        acc = acc + m * (1.0 / jnp.maximum(ctot, 1.0))
    hn = jnp.maximum(acc, 0.0)
    hb = hn.astype(BF16)
    for c, o in enumerate((o0, o1)):
        o[...] = hb[:, c * CHUNK:(c + 1) * CHUNK]
    if is_last:
        hm = hn * maskf[:, 0:1]
        olog[...] = jnp.dot(hm, wcls[...], preferred_element_type=F32) + bcls[...]


def _layer(hc, aggs, cnt, wroot, wrel, bconv, is_last,
           maskf=None, wcls=None, bcls=None):
    in_specs = (
        [pl.BlockSpec((BLK, CHUNK), lambda i: (i, 0))] * NCHUNK
        + [pl.BlockSpec((R, BLK, CHUNK), lambda i: (0, i, 0))] * NCHUNK
        + [pl.BlockSpec((NCORE, R, BLK, 8), lambda i: (0, 0, i, 0)),
           pl.BlockSpec((H, H), lambda i: (0, 0)),
           pl.BlockSpec((R, H, H), lambda i: (0, 0, 0)),
           pl.BlockSpec((1, H), lambda i: (0, 0))])
    out_specs = [pl.BlockSpec((BLK, CHUNK), lambda i: (i, 0))] * NCHUNK
    out_shape = [jax.ShapeDtypeStruct((N, CHUNK), BF16)] * NCHUNK
    args = list(hc) + list(aggs) + [cnt, wroot, wrel, bconv]
    if is_last:
        in_specs += [pl.BlockSpec((BLK, 8), lambda i: (i, 0)),
                     pl.BlockSpec((H, H), lambda i: (0, 0)),
                     pl.BlockSpec((1, H), lambda i: (0, 0))]
        out_specs.append(pl.BlockSpec((BLK, H), lambda i: (i, 0)))
        out_shape.append(jax.ShapeDtypeStruct((N, H), F32))
        args += [maskf, wcls, bcls]
    return pl.pallas_call(
        functools.partial(_layer_body, is_last),
        grid=(GRID,),
        in_specs=in_specs,
        out_specs=out_specs,
        out_shape=out_shape,
    )(*args)


def kernel(x_address, x_transaction, edge_index, edge_type, train_mask,
           W_in_a, b_in_a, W_in_t, b_in_t, W_rel, W_root, b_conv,
           W_cls, b_cls):
    x = jnp.concatenate([x_address, x_transaction], axis=0)
    src = edge_index[0]
    dst = edge_index[1]
    pad = E_PAD - E
    sidx = jnp.concatenate([src, jnp.zeros((pad,), jnp.int32)])
    didx = jnp.concatenate([edge_type * N + dst,
                            jnp.full((pad,), TRASH, jnp.int32)])

    zeros64 = jnp.zeros((ZR, CHUNK), BF16)
    zeros8 = jnp.zeros((ROWS_PER_SUB, 8), F32)
    ones8 = jnp.ones((GRP, 8), F32)
    maskf = jnp.concatenate(
        [train_mask[:N_ADDR].astype(F32), jnp.zeros((N - N_ADDR,), F32)]
    )[:, None] * jnp.ones((1, 8), F32)
    wcls_pad = jnp.pad(W_cls, ((0, 0), (0, H - C)))
    bcls_pad = jnp.pad(b_cls, (0, H - C))[None, :]

    sidx2 = sidx.reshape(E_PAD // GRP, GRP)
    didx2 = didx.reshape(E_PAD // GRP, GRP)

    hc = _proj(x, W_in_a, W_in_t, b_in_a[None, :], b_in_t[None, :])
    cnt_raw = _counts(didx, ones8, zeros8)
    cnt = jnp.stack([cnt_raw[c * RN_PAD:c * RN_PAD + RN]
                     for c in range(NCORE)]).reshape(NCORE, R, N, 8)

    logits = None
    for l in range(W_root.shape[0]):
        is_last = l == W_root.shape[0] - 1
        aggs = _aggregate(hc, sidx2, didx2, zeros64)
        aggs = [a[:RN].reshape(R, N, CHUNK) for a in aggs]
        res = _layer(hc, aggs, cnt, W_root[l], W_rel[l], b_conv[l][None, :],
                     is_last, maskf=maskf if is_last else None,
                     wcls=wcls_pad if is_last else None,
                     bcls=bcls_pad if is_last else None)
        if is_last:
            hc, logits = res[:NCHUNK], res[NCHUNK]
        else:
            hc = res
    return logits[:N_ADDR, :C]


# speedup vs baseline: 18.5153x; 1.0392x over previous
import functools

import jax
import jax.numpy as jnp
from jax import lax
from jax.experimental import pallas as pl
from jax.experimental.pallas import tpu as pltpu
from jax.experimental.pallas import tpu_sc as plsc

F32 = jnp.float32
BF16 = jnp.bfloat16

N_ADDR = 6000
N_TX = 4000
N = N_ADDR + N_TX
E = 320000
H = 128
R = 4
C = 2

NSUB = 16
NCORE = 2
GRP = 128
NGRP = 160
KGRP = 8
E_PAD = NSUB * NGRP * GRP
CGRP = E_PAD // (NSUB * NCORE * GRP)

RN = R * N
T = 40960
TRASH = RN
ROWS_PER_SUB = T // NSUB
OUT_PER_SUB = 2504
RN_PAD = NSUB * OUT_PER_SUB
CHUNK = 64
NCHUNK = H // CHUNK
ZR = 160

BLK = 1000
GRID = N // BLK
ABLK = N_ADDR // BLK


def _proj_body(x_ref, wa_ref, wt_ref, ba_ref, bt_ref, o0, o1):
    i = pl.program_id(0)
    use_a = i < ABLK
    w = jnp.where(use_a, wa_ref[...], wt_ref[...])
    b = jnp.where(use_a, ba_ref[...], bt_ref[...])
    h = (jnp.dot(x_ref[...], w, preferred_element_type=F32) + b).astype(BF16)
    for c, o in enumerate((o0, o1)):
        o[...] = h[:, c * CHUNK:(c + 1) * CHUNK]


def _proj(x, wa, wt, ba, bt):
    return pl.pallas_call(
        _proj_body,
        grid=(GRID,),
        in_specs=[
            pl.BlockSpec((BLK, H), lambda i: (i, 0)),
            pl.BlockSpec((H, H), lambda i: (0, 0)),
            pl.BlockSpec((H, H), lambda i: (0, 0)),
            pl.BlockSpec((1, H), lambda i: (0, 0)),
            pl.BlockSpec((1, H), lambda i: (0, 0)),
        ],
        out_specs=[pl.BlockSpec((BLK, CHUNK), lambda i: (i, 0))] * NCHUNK,
        out_shape=[jax.ShapeDtypeStruct((N, CHUNK), BF16)] * NCHUNK,
    )(x, wa, wt, ba, bt)


def _cnt_body(didx, ones_h, zeros_h, out, cnt_sp, dbuf, ones_v, zbuf):
    c = lax.axis_index("c")
    s = lax.axis_index("s")
    pltpu.sync_copy(ones_h, ones_v)
    pltpu.sync_copy(zeros_h, zbuf)
    pltpu.sync_copy(zbuf, cnt_sp.at[pl.ds(s * ROWS_PER_SUB, ROWS_PER_SUB)])
    plsc.subcore_barrier()
    w = c * NSUB + s

    def body(g, carry):
        off = (w * CGRP + g) * GRP
        pltpu.sync_copy(didx.at[pl.ds(off, GRP)], dbuf)
        pltpu.sync_copy(ones_v, cnt_sp.at[dbuf], add=True)
        return carry

    lax.fori_loop(0, CGRP, body, 0)
    plsc.subcore_barrier()
    pltpu.sync_copy(
        cnt_sp.at[pl.ds(s * OUT_PER_SUB, OUT_PER_SUB)],
        out.at[pl.ds(c * RN_PAD + s * OUT_PER_SUB, OUT_PER_SUB)])


def _counts(didx, ones8, zeros8):
    k = pl.kernel(
        _cnt_body,
        out_type=jax.ShapeDtypeStruct((NCORE * RN_PAD, 8), F32),
        mesh=plsc.VectorSubcoreMesh(core_axis_name="c", subcore_axis_name="s"),
        compiler_params=pltpu.CompilerParams(use_tc_tiling_on_sc=False),
        scratch_types=[
            pltpu.VMEM_SHARED((T, 8), F32),
            pltpu.VMEM((GRP,), jnp.int32),
            pltpu.VMEM((GRP, 8), F32),
            pltpu.VMEM((ROWS_PER_SUB, 8), F32),
        ],
    )
    return k(didx, ones8, zeros8)


def _agg_body(h0, h1, sidx, didx, zeros_h, a0, a1,
              agg_sp, sbuf, dbuf, rows, zbuf, gsem, ssem):
    c = lax.axis_index("c")
    s = lax.axis_index("s")
    pltpu.sync_copy(zeros_h, zbuf)

    def do_pass(table, out):
        for z in range(ROWS_PER_SUB // ZR):
            pltpu.sync_copy(
                zbuf, agg_sp.at[pl.ds(s * ROWS_PER_SUB + z * ZR, ZR)])
        plsc.subcore_barrier()

        def body(g, carry):
            row0 = s * NGRP + g * KGRP
            pltpu.sync_copy(sidx.at[pl.ds(row0, KGRP)], sbuf)
            pltpu.sync_copy(didx.at[pl.ds(row0, KGRP)], dbuf)
            gd = [pltpu.async_copy(table.at[sbuf.at[j]],
                                   rows.at[pl.ds(j * GRP, GRP)], gsem)
                  for j in range(KGRP)]
            sd = []
            for j in range(KGRP):
                gd[j].wait()
                sd.append(pltpu.async_copy(rows.at[pl.ds(j * GRP, GRP)],
                                           agg_sp.at[dbuf.at[j]], ssem,
                                           add=True))
            for d in sd:
                d.wait()
            return carry

        lax.fori_loop(0, NGRP // KGRP, body, 0)
        plsc.subcore_barrier()
        pltpu.sync_copy(
            agg_sp.at[pl.ds(s * OUT_PER_SUB, OUT_PER_SUB)],
            out.at[pl.ds(s * OUT_PER_SUB, OUT_PER_SUB)])

    @pl.when(c == 0)
    def _():
        do_pass(h0, a0)

    @pl.when(c == 1)
    def _():
        do_pass(h1, a1)


def _aggregate(hc, sidx, didx, zeros64):
    k = pl.kernel(
        _agg_body,
        out_type=[jax.ShapeDtypeStruct((RN_PAD, CHUNK), BF16)] * NCHUNK,
        mesh=plsc.VectorSubcoreMesh(core_axis_name="c", subcore_axis_name="s"),
        compiler_params=pltpu.CompilerParams(use_tc_tiling_on_sc=False),
        scratch_types=[
            pltpu.VMEM_SHARED((T, CHUNK), BF16),
            pltpu.VMEM((KGRP, GRP), jnp.int32),
            pltpu.VMEM((KGRP, GRP), jnp.int32),
            pltpu.VMEM((KGRP * GRP, CHUNK), BF16),
            pltpu.VMEM((ZR, CHUNK), BF16),
            pltpu.SemaphoreType.DMA,
            pltpu.SemaphoreType.DMA,
        ],
    )
    return k(hc[0], hc[1], sidx, didx, zeros64)


def _root_body(h0, h1, wroot_ref, bconv_ref, out):
    h = jnp.concatenate([h0[...], h1[...]], axis=1).astype(F32)
    out[...] = jnp.dot(h, wroot_ref[...], preferred_element_type=F32) \
        + bconv_ref[...]


def _root(hc, wroot, bconv):
    return pl.pallas_call(
        _root_body,
        grid=(GRID,),
        in_specs=[pl.BlockSpec((BLK, CHUNK), lambda i: (i, 0))] * NCHUNK
        + [pl.BlockSpec((H, H), lambda i: (0, 0)),
           pl.BlockSpec((1, H), lambda i: (0, 0))],
        out_specs=pl.BlockSpec((BLK, H), lambda i: (i, 0)),
        out_shape=jax.ShapeDtypeStruct((N, H), F32),
    )(hc[0], hc[1], wroot, bconv)


def _combine_body(is_last, *refs):
    if is_last:
        (root, a00, a01, a02, a03, a10, a11, a12, a13, cnt, wrel,
         maskf, wcls, bcls, o0, o1, olog) = refs
    else:
        (root, a00, a01, a02, a03, a10, a11, a12, a13, cnt, wrel,
         o0, o1) = refs
    av = ((a00, a01, a02, a03), (a10, a11, a12, a13))
    acc = root[...]
    wr = wrel[...]
    cn = cnt[...]
    for r in range(R):
        ar = jnp.concatenate([av[c][r][...] for c in range(NCHUNK)],
                             axis=1).astype(F32)
        m = jnp.dot(ar, wr[r], preferred_element_type=F32)
        ctot = cn[0, r, :, 0:1] + cn[1, r, :, 0:1]
        acc = acc + m * (1.0 / jnp.maximum(ctot, 1.0))
    hn = jnp.maximum(acc, 0.0)
    hb = hn.astype(BF16)
    for c, o in enumerate((o0, o1)):
        o[...] = hb[:, c * CHUNK:(c + 1) * CHUNK]
    if is_last:
        hm = hn * maskf[:, 0:1]
        olog[...] = jnp.dot(hm, wcls[...], preferred_element_type=F32) + bcls[...]


def _combine(root, aggs_raw, cnt, wrel, is_last,
             maskf=None, wcls=None, bcls=None):
    agg_specs = [pl.BlockSpec((BLK, CHUNK), lambda i, rr=r: (10 * rr + i, 0))
                 for c in range(NCHUNK) for r in range(R)]
    agg_args = [aggs_raw[c] for c in range(NCHUNK) for r in range(R)]
    in_specs = (
        [pl.BlockSpec((BLK, H), lambda i: (i, 0))]
        + agg_specs
        + [pl.BlockSpec((NCORE, R, BLK, 8), lambda i: (0, 0, i, 0)),
           pl.BlockSpec((R, H, H), lambda i: (0, 0, 0))])
    out_specs = [pl.BlockSpec((BLK, CHUNK), lambda i: (i, 0))] * NCHUNK
    out_shape = [jax.ShapeDtypeStruct((N, CHUNK), BF16)] * NCHUNK
    args = [root] + agg_args + [cnt, wrel]
    if is_last:
        in_specs += [pl.BlockSpec((BLK, 8), lambda i: (i, 0)),
                     pl.BlockSpec((H, H), lambda i: (0, 0)),
                     pl.BlockSpec((1, H), lambda i: (0, 0))]
        out_specs.append(pl.BlockSpec((BLK, H), lambda i: (i, 0)))
        out_shape.append(jax.ShapeDtypeStruct((N, H), F32))
        args += [maskf, wcls, bcls]
    return pl.pallas_call(
        functools.partial(_combine_body, is_last),
        grid=(GRID,),
        in_specs=in_specs,
        out_specs=out_specs,
        out_shape=out_shape,
    )(*args)


def kernel(x_address, x_transaction, edge_index, edge_type, train_mask,
           W_in_a, b_in_a, W_in_t, b_in_t, W_rel, W_root, b_conv,
           W_cls, b_cls):
    x = jnp.concatenate([x_address, x_transaction], axis=0)
    src = edge_index[0]
    dst = edge_index[1]
    pad = E_PAD - E
    sidx = jnp.concatenate([src, jnp.zeros((pad,), jnp.int32)])
    didx = jnp.concatenate([edge_type * N + dst,
                            jnp.full((pad,), TRASH, jnp.int32)])

    zeros64 = jnp.zeros((ZR, CHUNK), BF16)
    zeros8 = jnp.zeros((ROWS_PER_SUB, 8), F32)
    ones8 = jnp.ones((GRP, 8), F32)
    maskf = jnp.concatenate(
        [train_mask[:N_ADDR].astype(F32), jnp.zeros((N - N_ADDR,), F32)]
    )[:, None] * jnp.ones((1, 8), F32)
    wcls_pad = jnp.pad(W_cls, ((0, 0), (0, H - C)))
    bcls_pad = jnp.pad(b_cls, (0, H - C))[None, :]

    sidx2 = sidx.reshape(E_PAD // GRP, GRP)
    didx2 = didx.reshape(E_PAD // GRP, GRP)

    hc = _proj(x, W_in_a, W_in_t, b_in_a[None, :], b_in_t[None, :])
    cnt_raw = _counts(didx, ones8, zeros8)
    cnt = jnp.stack([cnt_raw[c * RN_PAD:c * RN_PAD + RN]
                     for c in range(NCORE)]).reshape(NCORE, R, N, 8)

    logits = None
    for l in range(W_root.shape[0]):
        is_last = l == W_root.shape[0] - 1
        aggs_raw = _aggregate(hc, sidx2, didx2, zeros64)
        root = _root(hc, W_root[l], b_conv[l][None, :])
        res = _combine(root, aggs_raw, cnt, W_rel[l], is_last,
                       maskf=maskf if is_last else None,
                       wcls=wcls_pad if is_last else None,
                       bcls=bcls_pad if is_last else None)
        if is_last:
            hc, logits = res[:NCHUNK], res[NCHUNK]
        else:
            hc = res
    return logits[:N_ADDR, :C]


# speedup vs baseline: 18.5792x; 1.0034x over previous
import functools

import jax
import jax.numpy as jnp
from jax import lax
from jax.experimental import pallas as pl
from jax.experimental.pallas import tpu as pltpu
from jax.experimental.pallas import tpu_sc as plsc

F32 = jnp.float32
BF16 = jnp.bfloat16

N_ADDR = 6000
N_TX = 4000
N = N_ADDR + N_TX
E = 320000
H = 128
R = 4
C = 2

NSUB = 16
NCORE = 2
GRP = 128
NGRP = 160
KGRP = 8
NBLK = NGRP // KGRP
E_PAD = NSUB * NGRP * GRP
CGRP = E_PAD // (NSUB * NCORE * GRP)

RN = R * N
T = 40960
TRASH = RN
ROWS_PER_SUB = T // NSUB
OUT_PER_SUB = 2504
RN_PAD = NSUB * OUT_PER_SUB
CHUNK = 64
NCHUNK = H // CHUNK
ZR = 160

BLK = 1000
GRID = N // BLK
ABLK = N_ADDR // BLK


def _proj_body(x_ref, wa_ref, wt_ref, ba_ref, bt_ref, o0, o1):
    i = pl.program_id(0)
    use_a = i < ABLK
    w = jnp.where(use_a, wa_ref[...], wt_ref[...])
    b = jnp.where(use_a, ba_ref[...], bt_ref[...])
    h = (jnp.dot(x_ref[...], w, preferred_element_type=F32) + b).astype(BF16)
    for c, o in enumerate((o0, o1)):
        o[...] = h[:, c * CHUNK:(c + 1) * CHUNK]


def _proj(x, wa, wt, ba, bt):
    return pl.pallas_call(
        _proj_body,
        grid=(GRID,),
        in_specs=[
            pl.BlockSpec((BLK, H), lambda i: (i, 0)),
            pl.BlockSpec((H, H), lambda i: (0, 0)),
            pl.BlockSpec((H, H), lambda i: (0, 0)),
            pl.BlockSpec((1, H), lambda i: (0, 0)),
            pl.BlockSpec((1, H), lambda i: (0, 0)),
        ],
        out_specs=[pl.BlockSpec((BLK, CHUNK), lambda i: (i, 0))] * NCHUNK,
        out_shape=[jax.ShapeDtypeStruct((N, CHUNK), BF16)] * NCHUNK,
    )(x, wa, wt, ba, bt)


def _cnt_body(didx, ones_h, zeros_h, out, cnt_sp, dbuf, ones_v, zbuf):
    c = lax.axis_index("c")
    s = lax.axis_index("s")
    pltpu.sync_copy(ones_h, ones_v)
    pltpu.sync_copy(zeros_h, zbuf)
    pltpu.sync_copy(zbuf, cnt_sp.at[pl.ds(s * ROWS_PER_SUB, ROWS_PER_SUB)])
    plsc.subcore_barrier()
    w = c * NSUB + s

    def body(g, carry):
        off = (w * CGRP + g) * GRP
        pltpu.sync_copy(didx.at[pl.ds(off, GRP)], dbuf)
        pltpu.sync_copy(ones_v, cnt_sp.at[dbuf], add=True)
        return carry

    lax.fori_loop(0, CGRP, body, 0)
    plsc.subcore_barrier()
    pltpu.sync_copy(
        cnt_sp.at[pl.ds(s * OUT_PER_SUB, OUT_PER_SUB)],
        out.at[pl.ds(c * RN_PAD + s * OUT_PER_SUB, OUT_PER_SUB)])


def _counts(didx, ones8, zeros8):
    k = pl.kernel(
        _cnt_body,
        out_type=jax.ShapeDtypeStruct((NCORE * RN_PAD, 8), F32),
        mesh=plsc.VectorSubcoreMesh(core_axis_name="c", subcore_axis_name="s"),
        compiler_params=pltpu.CompilerParams(use_tc_tiling_on_sc=False),
        scratch_types=[
            pltpu.VMEM_SHARED((T, 8), F32),
            pltpu.VMEM((GRP,), jnp.int32),
            pltpu.VMEM((GRP, 8), F32),
            pltpu.VMEM((ROWS_PER_SUB, 8), F32),
        ],
    )
    return k(didx, ones8, zeros8)


def _agg_body(h0, h1, icat, zeros_h, a0, a1,
              agg_sp, ibuf, rows, zbuf, gsem, ssem):
    c = lax.axis_index("c")
    s = lax.axis_index("s")
    pltpu.sync_copy(zeros_h, zbuf)

    def do_pass(table, out):
        for z in range(ROWS_PER_SUB // ZR):
            pltpu.sync_copy(
                zbuf, agg_sp.at[pl.ds(s * ROWS_PER_SUB + z * ZR, ZR)])
        plsc.subcore_barrier()

        def body(g, carry):
            row0 = (s * NBLK + g) * 2 * KGRP
            pltpu.sync_copy(icat.at[pl.ds(row0, 2 * KGRP)], ibuf)
            gd = [pltpu.async_copy(table.at[ibuf.at[j]],
                                   rows.at[pl.ds(j * GRP, GRP)], gsem)
                  for j in range(KGRP)]
            sd = []
            for j in range(KGRP):
                gd[j].wait()
                sd.append(pltpu.async_copy(rows.at[pl.ds(j * GRP, GRP)],
                                           agg_sp.at[ibuf.at[KGRP + j]],
                                           ssem, add=True))
            for d in sd:
                d.wait()
            return carry

        lax.fori_loop(0, NGRP // KGRP, body, 0)
        plsc.subcore_barrier()
        pltpu.sync_copy(
            agg_sp.at[pl.ds(s * OUT_PER_SUB, OUT_PER_SUB)],
            out.at[pl.ds(s * OUT_PER_SUB, OUT_PER_SUB)])

    @pl.when(c == 0)
    def _():
        do_pass(h0, a0)

    @pl.when(c == 1)
    def _():
        do_pass(h1, a1)


def _aggregate(hc, icat, zeros64):
    k = pl.kernel(
        _agg_body,
        out_type=[jax.ShapeDtypeStruct((RN_PAD, CHUNK), BF16)] * NCHUNK,
        mesh=plsc.VectorSubcoreMesh(core_axis_name="c", subcore_axis_name="s"),
        compiler_params=pltpu.CompilerParams(use_tc_tiling_on_sc=False),
        scratch_types=[
            pltpu.VMEM_SHARED((T, CHUNK), BF16),
            pltpu.VMEM((2 * KGRP, GRP), jnp.int32),
            pltpu.VMEM((KGRP * GRP, CHUNK), BF16),
            pltpu.VMEM((ZR, CHUNK), BF16),
            pltpu.SemaphoreType.DMA,
            pltpu.SemaphoreType.DMA,
        ],
    )
    return k(hc[0], hc[1], icat, zeros64)


def _root_body(h0, h1, wroot_ref, bconv_ref, out):
    h = jnp.concatenate([h0[...], h1[...]], axis=1).astype(F32)
    out[...] = jnp.dot(h, wroot_ref[...], preferred_element_type=F32) \
        + bconv_ref[...]


def _root(hc, wroot, bconv):
    return pl.pallas_call(
        _root_body,
        grid=(GRID,),
        in_specs=[pl.BlockSpec((BLK, CHUNK), lambda i: (i, 0))] * NCHUNK
        + [pl.BlockSpec((H, H), lambda i: (0, 0)),
           pl.BlockSpec((1, H), lambda i: (0, 0))],
        out_specs=pl.BlockSpec((BLK, H), lambda i: (i, 0)),
        out_shape=jax.ShapeDtypeStruct((N, H), F32),
    )(hc[0], hc[1], wroot, bconv)


def _combine_body(is_last, *refs):
    if is_last:
        (root, a00, a01, a02, a03, a10, a11, a12, a13, cnt, wrel,
         maskf, wcls, bcls, o0, o1, olog) = refs
    else:
        (root, a00, a01, a02, a03, a10, a11, a12, a13, cnt, wrel,
         o0, o1) = refs
    av = ((a00, a01, a02, a03), (a10, a11, a12, a13))
    acc = root[...]
    wr = wrel[...]
    cn = cnt[...]
    for r in range(R):
        ar = jnp.concatenate([av[c][r][...] for c in range(NCHUNK)],
                             axis=1).astype(F32)
        m = jnp.dot(ar, wr[r], preferred_element_type=F32)
        ctot = cn[0, r, :, 0:1] + cn[1, r, :, 0:1]
        acc = acc + m * (1.0 / jnp.maximum(ctot, 1.0))
    hn = jnp.maximum(acc, 0.0)
    hb = hn.astype(BF16)
    for c, o in enumerate((o0, o1)):
        o[...] = hb[:, c * CHUNK:(c + 1) * CHUNK]
    if is_last:
        hm = hn * maskf[:, 0:1]
        olog[...] = jnp.dot(hm, wcls[...], preferred_element_type=F32) + bcls[...]


def _combine(root, aggs_raw, cnt, wrel, is_last,
             maskf=None, wcls=None, bcls=None):
    agg_specs = [pl.BlockSpec((BLK, CHUNK), lambda i, rr=r: (10 * rr + i, 0))
                 for c in range(NCHUNK) for r in range(R)]
    agg_args = [aggs_raw[c] for c in range(NCHUNK) for r in range(R)]
    in_specs = (
        [pl.BlockSpec((BLK, H), lambda i: (i, 0))]
        + agg_specs
        + [pl.BlockSpec((NCORE, R, BLK, 8), lambda i: (0, 0, i, 0)),
           pl.BlockSpec((R, H, H), lambda i: (0, 0, 0))])
    out_specs = [pl.BlockSpec((BLK, CHUNK), lambda i: (i, 0))] * NCHUNK
    out_shape = [jax.ShapeDtypeStruct((N, CHUNK), BF16)] * NCHUNK
    args = [root] + agg_args + [cnt, wrel]
    if is_last:
        in_specs += [pl.BlockSpec((BLK, 8), lambda i: (i, 0)),
                     pl.BlockSpec((H, H), lambda i: (0, 0)),
                     pl.BlockSpec((1, H), lambda i: (0, 0))]
        out_specs.append(pl.BlockSpec((BLK, H), lambda i: (i, 0)))
        out_shape.append(jax.ShapeDtypeStruct((N, H), F32))
        args += [maskf, wcls, bcls]
    return pl.pallas_call(
        functools.partial(_combine_body, is_last),
        grid=(GRID,),
        in_specs=in_specs,
        out_specs=out_specs,
        out_shape=out_shape,
    )(*args)


def kernel(x_address, x_transaction, edge_index, edge_type, train_mask,
           W_in_a, b_in_a, W_in_t, b_in_t, W_rel, W_root, b_conv,
           W_cls, b_cls):
    x = jnp.concatenate([x_address, x_transaction], axis=0)
    src = edge_index[0]
    dst = edge_index[1]
    pad = E_PAD - E
    sidx = jnp.concatenate([src, jnp.zeros((pad,), jnp.int32)])
    didx = jnp.concatenate([edge_type * N + dst,
                            jnp.full((pad,), TRASH, jnp.int32)])

    zeros64 = jnp.zeros((ZR, CHUNK), BF16)
    zeros8 = jnp.zeros((ROWS_PER_SUB, 8), F32)
    ones8 = jnp.ones((GRP, 8), F32)
    maskf = jnp.concatenate(
        [train_mask[:N_ADDR].astype(F32), jnp.zeros((N - N_ADDR,), F32)]
    )[:, None] * jnp.ones((1, 8), F32)
    wcls_pad = jnp.pad(W_cls, ((0, 0), (0, H - C)))
    bcls_pad = jnp.pad(b_cls, (0, H - C))[None, :]

    s4 = sidx.reshape(NSUB, NBLK, KGRP, GRP)
    d4 = didx.reshape(NSUB, NBLK, KGRP, GRP)
    icat = jnp.concatenate([s4, d4], axis=2).reshape(-1, GRP)

    hc = _proj(x, W_in_a, W_in_t, b_in_a[None, :], b_in_t[None, :])
    cnt_raw = _counts(didx, ones8, zeros8)
    cnt = jnp.stack([cnt_raw[c * RN_PAD:c * RN_PAD + RN]
                     for c in range(NCORE)]).reshape(NCORE, R, N, 8)

    logits = None
    for l in range(W_root.shape[0]):
        is_last = l == W_root.shape[0] - 1
        aggs_raw = _aggregate(hc, icat, zeros64)
        root = _root(hc, W_root[l], b_conv[l][None, :])
        res = _combine(root, aggs_raw, cnt, W_rel[l], is_last,
                       maskf=maskf if is_last else None,
                       wcls=wcls_pad if is_last else None,
                       bcls=bcls_pad if is_last else None)
        if is_last:
            hc, logits = res[:NCHUNK], res[NCHUNK]
        else:
            hc = res
    return logits[:N_ADDR, :C]


# speedup vs baseline: 19.2549x; 1.0364x over previous
import functools

import jax
import jax.numpy as jnp
from jax import lax
from jax.experimental import pallas as pl
from jax.experimental.pallas import tpu as pltpu
from jax.experimental.pallas import tpu_sc as plsc

F32 = jnp.float32
BF16 = jnp.bfloat16

N_ADDR = 6000
N_TX = 4000
N = N_ADDR + N_TX
E = 320000
H = 128
R = 4
C = 2

NSUB = 16
NCORE = 2
GRP = 128
NGRP = 160
KGRP = 8
NBLK = NGRP // KGRP
E_PAD = NSUB * NGRP * GRP
CGRP = E_PAD // (NSUB * NCORE * GRP)

RN = R * N
T = 40960
TRASH = RN
ROWS_PER_SUB = T // NSUB
OUT_PER_SUB = 2504
RN_PAD = NSUB * OUT_PER_SUB
CHUNK = 64
NCHUNK = H // CHUNK
ZR = 160

BLK = 1000
GRID = N // BLK
ABLK = N_ADDR // BLK


def _proj_body(x_ref, wa_ref, wt_ref, ba_ref, bt_ref, wroot_ref, bconv_ref,
               o0, o1, oroot):
    i = pl.program_id(0)
    use_a = i < ABLK
    w = jnp.where(use_a, wa_ref[...], wt_ref[...])
    b = jnp.where(use_a, ba_ref[...], bt_ref[...])
    h = jnp.dot(x_ref[...], w, preferred_element_type=F32) + b
    hb = h.astype(BF16)
    for c, o in enumerate((o0, o1)):
        o[...] = hb[:, c * CHUNK:(c + 1) * CHUNK]
    oroot[...] = jnp.dot(h, wroot_ref[...], preferred_element_type=F32) \
        + bconv_ref[...]


def _proj(x, wa, wt, ba, bt, wroot0, bconv0):
    return pl.pallas_call(
        _proj_body,
        grid=(GRID,),
        in_specs=[
            pl.BlockSpec((BLK, H), lambda i: (i, 0)),
            pl.BlockSpec((H, H), lambda i: (0, 0)),
            pl.BlockSpec((H, H), lambda i: (0, 0)),
            pl.BlockSpec((1, H), lambda i: (0, 0)),
            pl.BlockSpec((1, H), lambda i: (0, 0)),
            pl.BlockSpec((H, H), lambda i: (0, 0)),
            pl.BlockSpec((1, H), lambda i: (0, 0)),
        ],
        out_specs=[pl.BlockSpec((BLK, CHUNK), lambda i: (i, 0))] * NCHUNK
        + [pl.BlockSpec((BLK, H), lambda i: (i, 0))],
        out_shape=[jax.ShapeDtypeStruct((N, CHUNK), BF16)] * NCHUNK
        + [jax.ShapeDtypeStruct((N, H), F32)],
    )(x, wa, wt, ba, bt, wroot0, bconv0)


def _cnt_body(didx, ones_h, zeros_h, out, cnt_sp, dbuf, ones_v, zbuf):
    c = lax.axis_index("c")
    s = lax.axis_index("s")
    pltpu.sync_copy(ones_h, ones_v)
    pltpu.sync_copy(zeros_h, zbuf)
    pltpu.sync_copy(zbuf, cnt_sp.at[pl.ds(s * ROWS_PER_SUB, ROWS_PER_SUB)])
    plsc.subcore_barrier()
    w = c * NSUB + s

    def body(g, carry):
        off = (w * CGRP + g) * GRP
        pltpu.sync_copy(didx.at[pl.ds(off, GRP)], dbuf)
        pltpu.sync_copy(ones_v, cnt_sp.at[dbuf], add=True)
        return carry

    lax.fori_loop(0, CGRP, body, 0)
    plsc.subcore_barrier()
    pltpu.sync_copy(
        cnt_sp.at[pl.ds(s * OUT_PER_SUB, OUT_PER_SUB)],
        out.at[pl.ds(c * RN_PAD + s * OUT_PER_SUB, OUT_PER_SUB)])


def _counts(didx, ones8, zeros8):
    k = pl.kernel(
        _cnt_body,
        out_type=jax.ShapeDtypeStruct((NCORE * RN_PAD, 8), F32),
        mesh=plsc.VectorSubcoreMesh(core_axis_name="c", subcore_axis_name="s"),
        compiler_params=pltpu.CompilerParams(use_tc_tiling_on_sc=False),
        scratch_types=[
            pltpu.VMEM_SHARED((T, 8), F32),
            pltpu.VMEM((GRP,), jnp.int32),
            pltpu.VMEM((GRP, 8), F32),
            pltpu.VMEM((ROWS_PER_SUB, 8), F32),
        ],
    )
    return k(didx, ones8, zeros8)


def _agg_body(h0, h1, icat, zeros_h, a0, a1,
              agg_sp, ibuf, rows, zbuf, gsem, ssem):
    c = lax.axis_index("c")
    s = lax.axis_index("s")
    pltpu.sync_copy(zeros_h, zbuf)

    def do_pass(table, out):
        for z in range(ROWS_PER_SUB // ZR):
            pltpu.sync_copy(
                zbuf, agg_sp.at[pl.ds(s * ROWS_PER_SUB + z * ZR, ZR)])
        plsc.subcore_barrier()

        def body(g, carry):
            row0 = (s * NBLK + g) * 2 * KGRP
            pltpu.sync_copy(icat.at[pl.ds(row0, 2 * KGRP)], ibuf)
            gd = [pltpu.async_copy(table.at[ibuf.at[j]],
                                   rows.at[pl.ds(j * GRP, GRP)], gsem)
                  for j in range(KGRP)]
            sd = []
            for j in range(KGRP):
                gd[j].wait()
                sd.append(pltpu.async_copy(rows.at[pl.ds(j * GRP, GRP)],
                                           agg_sp.at[ibuf.at[KGRP + j]],
                                           ssem, add=True))
            for d in sd:
                d.wait()
            return carry

        lax.fori_loop(0, NGRP // KGRP, body, 0)
        plsc.subcore_barrier()
        pltpu.sync_copy(
            agg_sp.at[pl.ds(s * OUT_PER_SUB, OUT_PER_SUB)],
            out.at[pl.ds(s * OUT_PER_SUB, OUT_PER_SUB)])

    @pl.when(c == 0)
    def _():
        do_pass(h0, a0)

    @pl.when(c == 1)
    def _():
        do_pass(h1, a1)


def _aggregate(hc, icat, zeros64):
    k = pl.kernel(
        _agg_body,
        out_type=[jax.ShapeDtypeStruct((RN_PAD, CHUNK), BF16)] * NCHUNK,
        mesh=plsc.VectorSubcoreMesh(core_axis_name="c", subcore_axis_name="s"),
        compiler_params=pltpu.CompilerParams(use_tc_tiling_on_sc=False),
        scratch_types=[
            pltpu.VMEM_SHARED((T, CHUNK), BF16),
            pltpu.VMEM((2 * KGRP, GRP), jnp.int32),
            pltpu.VMEM((KGRP * GRP, CHUNK), BF16),
            pltpu.VMEM((ZR, CHUNK), BF16),
            pltpu.SemaphoreType.DMA,
            pltpu.SemaphoreType.DMA,
        ],
    )
    return k(hc[0], hc[1], icat, zeros64)


def _combine_body(is_last, *refs):
    if is_last:
        (root, a00, a01, a02, a03, a10, a11, a12, a13, cnt, wrel,
         maskf, wcls, bcls, o0, o1, olog) = refs
    else:
        (root, a00, a01, a02, a03, a10, a11, a12, a13, cnt, wrel,
         wroot_n, bconv_n, o0, o1, oroot) = refs
    av = ((a00, a01, a02, a03), (a10, a11, a12, a13))
    acc = root[...]
    wr = wrel[...]
    cn = cnt[...]
    for r in range(R):
        ar = jnp.concatenate([av[c][r][...] for c in range(NCHUNK)],
                             axis=1).astype(F32)
        m = jnp.dot(ar, wr[r], preferred_element_type=F32)
        ctot = cn[0, r, :, 0:1] + cn[1, r, :, 0:1]
        acc = acc + m * (1.0 / jnp.maximum(ctot, 1.0))
    hn = jnp.maximum(acc, 0.0)
    hb = hn.astype(BF16)
    for c, o in enumerate((o0, o1)):
        o[...] = hb[:, c * CHUNK:(c + 1) * CHUNK]
    if is_last:
        hm = hn * maskf[:, 0:1]
        olog[...] = jnp.dot(hm, wcls[...], preferred_element_type=F32) + bcls[...]
    else:
        oroot[...] = jnp.dot(hn, wroot_n[...], preferred_element_type=F32) \
            + bconv_n[...]


def _combine(root, aggs_raw, cnt, wrel, is_last,
             maskf=None, wcls=None, bcls=None, wroot_n=None, bconv_n=None):
    agg_specs = [pl.BlockSpec((BLK, CHUNK), lambda i, rr=r: (10 * rr + i, 0))
                 for c in range(NCHUNK) for r in range(R)]
    agg_args = [aggs_raw[c] for c in range(NCHUNK) for r in range(R)]
    in_specs = (
        [pl.BlockSpec((BLK, H), lambda i: (i, 0))]
        + agg_specs
        + [pl.BlockSpec((NCORE, R, BLK, 8), lambda i: (0, 0, i, 0)),
           pl.BlockSpec((R, H, H), lambda i: (0, 0, 0))])
    out_specs = [pl.BlockSpec((BLK, CHUNK), lambda i: (i, 0))] * NCHUNK
    out_shape = [jax.ShapeDtypeStruct((N, CHUNK), BF16)] * NCHUNK
    args = [root] + agg_args + [cnt, wrel]
    if is_last:
        in_specs += [pl.BlockSpec((BLK, 8), lambda i: (i, 0)),
                     pl.BlockSpec((H, H), lambda i: (0, 0)),
                     pl.BlockSpec((1, H), lambda i: (0, 0))]
        out_specs.append(pl.BlockSpec((BLK, H), lambda i: (i, 0)))
        out_shape.append(jax.ShapeDtypeStruct((N, H), F32))
        args += [maskf, wcls, bcls]
    else:
        in_specs += [pl.BlockSpec((H, H), lambda i: (0, 0)),
                     pl.BlockSpec((1, H), lambda i: (0, 0))]
        out_specs.append(pl.BlockSpec((BLK, H), lambda i: (i, 0)))
        out_shape.append(jax.ShapeDtypeStruct((N, H), F32))
        args += [wroot_n, bconv_n]
    return pl.pallas_call(
        functools.partial(_combine_body, is_last),
        grid=(GRID,),
        in_specs=in_specs,
        out_specs=out_specs,
        out_shape=out_shape,
    )(*args)


def kernel(x_address, x_transaction, edge_index, edge_type, train_mask,
           W_in_a, b_in_a, W_in_t, b_in_t, W_rel, W_root, b_conv,
           W_cls, b_cls):
    x = jnp.concatenate([x_address, x_transaction], axis=0)
    src = edge_index[0]
    dst = edge_index[1]
    pad = E_PAD - E
    sidx = jnp.concatenate([src, jnp.zeros((pad,), jnp.int32)])
    didx = jnp.concatenate([edge_type * N + dst,
                            jnp.full((pad,), TRASH, jnp.int32)])

    zeros64 = jnp.zeros((ZR, CHUNK), BF16)
    zeros8 = jnp.zeros((ROWS_PER_SUB, 8), F32)
    ones8 = jnp.ones((GRP, 8), F32)
    maskf = jnp.concatenate(
        [train_mask[:N_ADDR].astype(F32), jnp.zeros((N - N_ADDR,), F32)]
    )[:, None] * jnp.ones((1, 8), F32)
    wcls_pad = jnp.pad(W_cls, ((0, 0), (0, H - C)))
    bcls_pad = jnp.pad(b_cls, (0, H - C))[None, :]

    s4 = sidx.reshape(NSUB, NBLK, KGRP, GRP)
    d4 = didx.reshape(NSUB, NBLK, KGRP, GRP)
    icat = jnp.concatenate([s4, d4], axis=2).reshape(-1, GRP)

    *hc, root = _proj(x, W_in_a, W_in_t, b_in_a[None, :], b_in_t[None, :],
                      W_root[0], b_conv[0][None, :])
    cnt_raw = _counts(didx, ones8, zeros8)
    cnt = jnp.stack([cnt_raw[c * RN_PAD:c * RN_PAD + RN]
                     for c in range(NCORE)]).reshape(NCORE, R, N, 8)

    logits = None
    L = W_root.shape[0]
    for l in range(L):
        is_last = l == L - 1
        aggs_raw = _aggregate(hc, icat, zeros64)
        res = _combine(root, aggs_raw, cnt, W_rel[l], is_last,
                       maskf=maskf if is_last else None,
                       wcls=wcls_pad if is_last else None,
                       bcls=bcls_pad if is_last else None,
                       wroot_n=None if is_last else W_root[l + 1],
                       bconv_n=None if is_last else b_conv[l + 1][None, :])
        if is_last:
            hc, logits = res[:NCHUNK], res[NCHUNK]
        else:
            hc, root = res[:NCHUNK], res[NCHUNK]
    return logits[:N_ADDR, :C]
